# Initial kernel scaffold; baseline (speedup 1.0000x reference)
import jax
import jax.numpy as jnp
from jax.experimental import pallas as pl


def kernel(x, ln_src_g, ln_src_b, Wq, bq, Wk, Wv, bv, Ws, bs, Wg, bg, Wo, bo, ln_post_g, ln_post_b, ln_ffpre_g, ln_ffpre_b, W1, b1, W2, b2, ln_ffpost_g, ln_ffpost_b, edge_index):
    raise NotImplementedError("write your pallas kernel here")


# speedup vs baseline: 1.0128x; 1.0128x over previous
import functools

import jax
import jax.numpy as jnp
from jax.experimental import pallas as pl

N = 10000
E = 160000
HID = 256
H = 8
D = 64
HD = H * D

NB = 256
NP = 10240
GRID = NP // NB


def _ln(x, g, b):
    m = jnp.mean(x, axis=-1, keepdims=True)
    v = jnp.mean((x - m) ** 2, axis=-1, keepdims=True)
    return (x - m) * jax.lax.rsqrt(v + 1e-5) * g + b


def _pre_body(x_ref, lsg, lsb, wq, bq, wk, wv, bv, ws, bs,
              lfg, lfb, w1, b1, w2, b2, lpg, lpb,
              q_ref, k_ref, vt_ref, s_ref, xs_ref, ffp_ref):
    xb = x_ref[...]
    xs = _ln(xb, lsg[...], lsb[...])
    xs_ref[...] = xs
    q_ref[...] = jnp.dot(xs, wq[...], preferred_element_type=jnp.float32) + bq[...]
    k_ref[...] = jnp.dot(xs, wk[...], preferred_element_type=jnp.float32)
    v = jnp.dot(xs, wv[...], preferred_element_type=jnp.float32) + bv[...]
    for h in range(H):
        vt_ref[h, :, :] = v[:, h * D:(h + 1) * D]
    s_ref[...] = jnp.dot(xs, ws[...], preferred_element_type=jnp.float32) + bs[...]
    ffin = _ln(xb, lfg[...], lfb[...])
    ff1 = jnp.maximum(
        jnp.dot(ffin, w1[...], preferred_element_type=jnp.float32) + b1[...], 0.0)
    ff = jnp.dot(ff1, w2[...], preferred_element_type=jnp.float32) + b2[...]
    ffp_ref[...] = _ln(ff, lpg[...], lpb[...])


def _row_spec():
    return pl.BlockSpec((NB, HID), lambda i: (i, 0))


def _full(shape):
    return pl.BlockSpec(shape, lambda i: tuple(0 for _ in shape))


def _pre(x, ln_src_g, ln_src_b, Wq, bq, Wk, Wv, bv, Ws, bs,
         ln_ffpre_g, ln_ffpre_b, W1, b1, W2, b2, ln_ffpost_g, ln_ffpost_b):
    out_shapes = (
        jax.ShapeDtypeStruct((NP, HD), jnp.float32),
        jax.ShapeDtypeStruct((NP, HD), jnp.float32),
        jax.ShapeDtypeStruct((H, NP, D), jnp.float32),
        jax.ShapeDtypeStruct((NP, HD), jnp.float32),
        jax.ShapeDtypeStruct((NP, HID), jnp.float32),
        jax.ShapeDtypeStruct((NP, HID), jnp.float32),
    )
    in_specs = [
        _row_spec(),
        _full((HID,)), _full((HID,)),
        _full((HID, HD)), _full((HD,)),
        _full((HID, HD)),
        _full((HID, HD)), _full((HD,)),
        _full((HID, HD)), _full((HD,)),
        _full((HID,)), _full((HID,)),
        _full((HID, 4 * HID)), _full((4 * HID,)),
        _full((4 * HID, HID)), _full((HID,)),
        _full((HID,)), _full((HID,)),
    ]
    out_specs = (
        pl.BlockSpec((NB, HD), lambda i: (i, 0)),
        pl.BlockSpec((NB, HD), lambda i: (i, 0)),
        pl.BlockSpec((H, NB, D), lambda i: (0, i, 0)),
        pl.BlockSpec((NB, HD), lambda i: (i, 0)),
        _row_spec(),
        _row_spec(),
    )
    return pl.pallas_call(
        _pre_body, grid=(GRID,), in_specs=in_specs, out_specs=out_specs,
        out_shape=out_shapes,
    )(x, ln_src_g, ln_src_b, Wq, bq, Wk, Wv, bv, Ws, bs,
      ln_ffpre_g, ln_ffpre_b, W1, b1, W2, b2, ln_ffpost_g, ln_ffpost_b)


def _post_body(aggu_ref, xs_ref, s_ref, x_ref, ffp_ref,
               wg, bg, wo, bo, lpg, lpb, out_ref):
    den = jnp.maximum(aggu_ref[:, :, D], 1e-30)
    cols = []
    for h in range(H):
        cols.append(aggu_ref[h, :, :D] / den[h][:, None])
    agg = jnp.concatenate(cols, axis=-1)
    xs = xs_ref[...]
    g = jax.nn.sigmoid(
        jnp.dot(agg, wg[:HD, :], preferred_element_type=jnp.float32)
        + jnp.dot(xs, wg[HD:, :], preferred_element_type=jnp.float32) + bg[...])
    upd = agg + g * (s_ref[...] - agg)
    attn_out = jnp.dot(upd, wo[...], preferred_element_type=jnp.float32) + bo[...]
    out_ref[...] = x_ref[...] + _ln(attn_out, lpg[...], lpb[...])


def _post(aggu, xs, s, x, ffp, Wg, bg, Wo, bo, ln_post_g, ln_post_b):
    in_specs = [
        pl.BlockSpec((H, NB, D + 16), lambda i: (0, i, 0)),
        _row_spec(),
        pl.BlockSpec((NB, HD), lambda i: (i, 0)),
        _row_spec(),
        _row_spec(),
        _full((HD + HID, HD)), _full((HD,)),
        _full((HD, HID)), _full((HID,)),
        _full((HID,)), _full((HID,)),
    ]
    out = pl.pallas_call(
        _post_body, grid=(GRID,), in_specs=in_specs,
        out_specs=_row_spec(),
        out_shape=jax.ShapeDtypeStruct((NP, HID), jnp.float32),
    )(aggu, xs, s, x, ffp, Wg, bg, Wo, bo, ln_post_g, ln_post_b)
    return out + ffp


def kernel(x, ln_src_g, ln_src_b, Wq, bq, Wk, Wv, bv, Ws, bs, Wg, bg, Wo, bo,
           ln_post_g, ln_post_b, ln_ffpre_g, ln_ffpre_b, W1, b1, W2, b2,
           ln_ffpost_g, ln_ffpost_b, edge_index):
    xp = jnp.pad(x, ((0, NP - N), (0, 0)))
    q, k, vt, s, xs, ffp = _pre(
        xp, ln_src_g, ln_src_b, Wq, bq, Wk, Wv, bv, Ws, bs,
        ln_ffpre_g, ln_ffpre_b, W1, b1, W2, b2, ln_ffpost_g, ln_ffpost_b)

    src = edge_index[0]
    dst = edge_index[1]
    scale = D ** -0.5
    qh = q[:N].reshape(N, H, D)
    kh = k[:N].reshape(N, H, D)
    sim = jnp.sum(qh[dst] * kh[src], axis=-1) * scale
    m = jnp.max(sim, axis=0)
    m = jnp.maximum(m, 0.0)
    e = jnp.exp(sim - m)
    vh = jnp.transpose(vt[:, :N, :], (1, 0, 2))
    msg = vh[src] * e[..., None]
    payload = jnp.concatenate(
        [msg, e[..., None], jnp.zeros((E, H, 15), jnp.float32)], axis=-1)
    aggu = jax.ops.segment_sum(payload, dst, num_segments=N)
    aggu = jnp.pad(aggu, ((0, NP - N), (0, 0), (0, 0))).transpose(1, 0, 2)

    out = _post(aggu, xs, s, xp, ffp, Wg, bg, Wo, bo, ln_post_g, ln_post_b)
    return out[:N]


# speedup vs baseline: 2.9758x; 2.9383x over previous
import dataclasses
import functools

import jax
import jax.numpy as jnp
from jax import lax
from jax.experimental import pallas as pl
from jax.experimental.pallas import tpu as pltpu
from jax.experimental.pallas import tpu_sc as plsc

N = 10000
E = 160000
HID = 256
H = 8
D = 64
HD = H * D

NB = 256
NP = 10240
GRID = NP // NB

EP = 163840
C1 = 64
C2 = 64
W1E = EP // 32
W2E = EP // 16
AGW = 80

_mesh = plsc.VectorSubcoreMesh(core_axis_name="c", subcore_axis_name="s")

_sc_params = pltpu.CompilerParams()
if "needs_layout_passes" in pltpu.CompilerParams.__dataclass_fields__:
    _sc_params = dataclasses.replace(_sc_params, needs_layout_passes=False)
if "use_tc_tiling_on_sc" in pltpu.CompilerParams.__dataclass_fields__:
    _sc_params = dataclasses.replace(_sc_params, use_tc_tiling_on_sc=False)


def _ln(x, g, b):
    m = jnp.mean(x, axis=-1, keepdims=True)
    v = jnp.mean((x - m) ** 2, axis=-1, keepdims=True)
    return (x - m) * jax.lax.rsqrt(v + 1e-5) * g + b


def _pre_body(x_ref, lsg, lsb, wq, bq, wk, wv, bv, ws, bs,
              lfg, lfb, w1, b1, w2, b2, lpg, lpb,
              q_ref, k_ref, vt_ref, s_ref, xs_ref, ffp_ref):
    xb = x_ref[...]
    xs = _ln(xb, lsg[...], lsb[...])
    xs_ref[...] = xs
    q_ref[...] = jnp.dot(xs, wq[...], preferred_element_type=jnp.float32) + bq[...]
    k_ref[...] = jnp.dot(xs, wk[...], preferred_element_type=jnp.float32)
    v = jnp.dot(xs, wv[...], preferred_element_type=jnp.float32) + bv[...]
    for p in range(H // 2):
        vt_ref[p, :, :] = v[:, p * 2 * D:(p + 1) * 2 * D]
    s_ref[...] = jnp.dot(xs, ws[...], preferred_element_type=jnp.float32) + bs[...]
    ffin = _ln(xb, lfg[...], lfb[...])
    ff1 = jnp.maximum(
        jnp.dot(ffin, w1[...], preferred_element_type=jnp.float32) + b1[...], 0.0)
    ff = jnp.dot(ff1, w2[...], preferred_element_type=jnp.float32) + b2[...]
    ffp_ref[...] = _ln(ff, lpg[...], lpb[...])


def _row_spec():
    return pl.BlockSpec((NB, HID), lambda i: (i, 0))


def _full(shape):
    return pl.BlockSpec(shape, lambda i: tuple(0 for _ in shape))


def _pre(x, ln_src_g, ln_src_b, Wq, bq, Wk, Wv, bv, Ws, bs,
         ln_ffpre_g, ln_ffpre_b, W1, b1, W2, b2, ln_ffpost_g, ln_ffpost_b):
    out_shapes = (
        jax.ShapeDtypeStruct((NP, HD), jnp.float32),
        jax.ShapeDtypeStruct((NP, HD), jnp.float32),
        jax.ShapeDtypeStruct((H * NP, D), jnp.float32),
        jax.ShapeDtypeStruct((NP, HD), jnp.float32),
        jax.ShapeDtypeStruct((NP, HID), jnp.float32),
        jax.ShapeDtypeStruct((NP, HID), jnp.float32),
    )
    in_specs = [
        _row_spec(),
        _full((HID,)), _full((HID,)),
        _full((HID, HD)), _full((HD,)),
        _full((HID, HD)),
        _full((HID, HD)), _full((HD,)),
        _full((HID, HD)), _full((HD,)),
        _full((HID,)), _full((HID,)),
        _full((HID, 4 * HID)), _full((4 * HID,)),
        _full((4 * HID, HID)), _full((HID,)),
        _full((HID,)), _full((HID,)),
    ]

    def _vt_spec():
        return pl.BlockSpec((H, NB, D), lambda i: (0, i, 0))

    out_specs = (
        pl.BlockSpec((NB, HD), lambda i: (i, 0)),
        pl.BlockSpec((NB, HD), lambda i: (i, 0)),
        _vt_spec(),
        pl.BlockSpec((NB, HD), lambda i: (i, 0)),
        _row_spec(),
        _row_spec(),
    )

    def body(*refs):
        (x_ref, lsg, lsb, wq, bq, wk, wv, bv, ws, bs, lfg, lfb,
         w1, b1, w2, b2, lpg, lpb,
         q_ref, k_ref, vt_ref, s_ref, xs_ref, ffp_ref) = refs
        _pre_body(x_ref, lsg, lsb, wq, bq, wk, wv, bv, ws, bs,
                  lfg, lfb, w1, b1, w2, b2, lpg, lpb,
                  q_ref, k_ref, vt_ref, s_ref, xs_ref, ffp_ref)

    q, k, vt, s, xs, ffp = pl.pallas_call(
        body, grid=(GRID,), in_specs=in_specs,
        out_specs=(
            out_specs[0], out_specs[1],
            pl.BlockSpec((H // 2, NB, 2 * D), lambda i: (0, i, 0)),
            out_specs[3], out_specs[4], out_specs[5]),
        out_shape=(
            out_shapes[0], out_shapes[1],
            jax.ShapeDtypeStruct((H // 2, NP, 2 * D), jnp.float32),
            out_shapes[3], out_shapes[4], out_shapes[5]),
    )(x, ln_src_g, ln_src_b, Wq, bq, Wk, Wv, bv, Ws, bs,
      ln_ffpre_g, ln_ffpre_b, W1, b1, W2, b2, ln_ffpost_g, ln_ffpost_b)
    return q, k, vt, s, xs, ffp


@functools.partial(
    pl.kernel, mesh=_mesh, compiler_params=_sc_params,
    out_type=(jax.ShapeDtypeStruct((H * EP,), jnp.float32),
              jax.ShapeDtypeStruct((32, H, 16), jnp.float32)),
    scratch_types=[
        pltpu.VMEM((C1,), jnp.int32),
        pltpu.VMEM((C1,), jnp.int32),
        pltpu.VMEM((C1, HD), jnp.float32),
        pltpu.VMEM((C1, HD), jnp.float32),
        pltpu.VMEM((H, C1), jnp.float32),
        pltpu.VMEM((H, 16), jnp.float32),
        pltpu.SemaphoreType.DMA,
        pltpu.SemaphoreType.DMA,
        pltpu.SemaphoreType.DMA,
    ])
def _sim_kernel(q_hbm, k_hbm, src_hbm, dst_hbm, sim_hbm, maxp_hbm,
                src_v, dst_v, qd_v, ks_v, sim_v, max_v, sem1, sem2, sem3):
    cid = lax.axis_index("c")
    sid = lax.axis_index("s")
    w = sid * 2 + cid
    scale = D ** -0.5
    for h in range(H):
        max_v[h, :] = jnp.full((16,), -3e38, jnp.float32)

    @pl.loop(0, W1E // C1)
    def _chunk(j):
        base = w * W1E + j * C1
        pltpu.sync_copy(src_hbm.at[pl.ds(base, C1)], src_v)
        pltpu.sync_copy(dst_hbm.at[pl.ds(base, C1)], dst_v)
        cp1 = pltpu.async_copy(q_hbm.at[dst_v], qd_v, sem1)
        cp2 = pltpu.async_copy(k_hbm.at[src_v], ks_v, sem2)
        cp1.wait()
        cp2.wait()
        for g in range(C1 // 16):
            eidx = lax.iota(jnp.int32, 16) + g * 16

            def dbody(dd, accs):
                out = []
                for h in range(H):
                    col = jnp.full((16,), h * D, jnp.int32) + dd
                    qg = plsc.load_gather(qd_v, [eidx, col])
                    kg = plsc.load_gather(ks_v, [eidx, col])
                    out.append(accs[h] + qg * kg)
                return tuple(out)

            accs = lax.fori_loop(
                0, D, dbody,
                tuple(jnp.zeros((16,), jnp.float32) for _ in range(H)))
            for h in range(H):
                simh = accs[h] * scale
                sim_v[h, pl.ds(g * 16, 16)] = simh
                max_v[h, :] = jnp.maximum(max_v[h, :], simh)
        cps = [pltpu.async_copy(sim_v.at[h],
                                sim_hbm.at[pl.ds(h * EP + base, C1)], sem3)
               for h in range(H)]
        for cp in cps:
            cp.wait()

    pltpu.sync_copy(max_v, maxp_hbm.at[w])


PW = 2 * AGW

@functools.partial(
    pl.kernel, mesh=_mesh, compiler_params=_sc_params,
    out_type=jax.ShapeDtypeStruct((H // 2 * NP, PW), jnp.float32),
    scratch_types=[
        pltpu.VMEM_SHARED((NP, PW), jnp.float32),
        pltpu.VMEM((C2,), jnp.int32),
        pltpu.VMEM((C2,), jnp.int32),
        pltpu.VMEM((C2,), jnp.int32),
        pltpu.VMEM((C2, 2 * D), jnp.float32),
        pltpu.VMEM((C2,), jnp.float32),
        pltpu.VMEM((C2,), jnp.float32),
        pltpu.VMEM((C2, PW), jnp.float32),
        pltpu.VMEM((H * 16,), jnp.float32),
        pltpu.SemaphoreType.DMA,
    ])
def _agg_kernel(vt_hbm, sim_hbm, src_hbm, dst_hbm, mrep_hbm, out_hbm,
                agg, src_v, dst_v, sidx_v, vs_v, e0_v, e1_v, msg_v, m_v, sem):
    cid = lax.axis_index("c")
    sid = lax.axis_index("s")
    pltpu.sync_copy(mrep_hbm, m_v)
    zero16 = jnp.zeros((16,), jnp.float32)

    def run_core(pairs):
        for p in pairs:
            h0 = 2 * p
            h1 = 2 * p + 1
            @pl.loop(0, C2)
            def _zrow(i):
                for t in range(PW // 16):
                    msg_v[i, pl.ds(t * 16, 16)] = zero16
            trow = sid * (NP // 16)
            for b in range(NP // 16 // C2):
                pltpu.sync_copy(msg_v, agg.at[pl.ds(trow + b * C2, C2)])
            plsc.subcore_barrier()

            @pl.loop(0, W2E // C2)
            def _chunk(j):
                base = sid * W2E + j * C2
                pltpu.sync_copy(src_hbm.at[pl.ds(base, C2)], src_v)
                pltpu.sync_copy(dst_hbm.at[pl.ds(base, C2)], dst_v)
                pltpu.sync_copy(sim_hbm.at[pl.ds(h0 * EP + base, C2)], e0_v)
                pltpu.sync_copy(sim_hbm.at[pl.ds(h1 * EP + base, C2)], e1_v)
                for g in range(C2 // 16):
                    sidx_v[pl.ds(g * 16, 16)] = (
                        src_v[pl.ds(g * 16, 16)] + p * NP)
                pltpu.async_copy(vt_hbm.at[sidx_v], vs_v, sem).wait()
                m0 = m_v[pl.ds(h0 * 16, 16)]
                m1 = m_v[pl.ds(h1 * 16, 16)]
                for g in range(C2 // 16):
                    eidx = lax.iota(jnp.int32, 16) + g * 16
                    e0 = jnp.exp(e0_v[pl.ds(g * 16, 16)] - m0)
                    e1 = jnp.exp(e1_v[pl.ds(g * 16, 16)] - m1)
                    plsc.store_scatter(
                        msg_v, [eidx, jnp.full((16,), D, jnp.int32)], e0)
                    plsc.store_scatter(
                        msg_v, [eidx, jnp.full((16,), AGW + D, jnp.int32)], e1)

                    def dbody(dd, _):
                        c0 = jnp.full((16,), 0, jnp.int32) + dd
                        c1 = jnp.full((16,), D, jnp.int32) + dd
                        v0 = plsc.load_gather(vs_v, [eidx, c0])
                        v1 = plsc.load_gather(vs_v, [eidx, c1])
                        plsc.store_scatter(msg_v, [eidx, c0], v0 * e0)
                        plsc.store_scatter(
                            msg_v, [eidx, jnp.full((16,), AGW, jnp.int32) + dd],
                            v1 * e1)
                        return 0

                    lax.fori_loop(0, D, dbody, 0)
                pltpu.sync_copy(msg_v, agg.at[dst_v], add=True)
            plsc.subcore_barrier()
            nrow = NP // 16
            pltpu.sync_copy(agg.at[pl.ds(trow, nrow)],
                            out_hbm.at[pl.ds(p * NP + trow, nrow)])
            plsc.subcore_barrier()

    @pl.when(cid == 0)
    def _c0():
        run_core((0, 1))

    @pl.when(cid == 1)
    def _c1():
        run_core((2, 3))


def _post_body(aggu_ref, xs_ref, s_ref, x_ref, ffp_ref,
               wg, bg, wo, bo, lpg, lpb, out_ref):
    cols = []
    for h in range(H):
        p, o = h // 2, (h % 2) * AGW
        den = jnp.maximum(aggu_ref[p, :, o + D], 1e-30)
        cols.append(aggu_ref[p, :, o:o + D] / den[:, None])
    agg = jnp.concatenate(cols, axis=-1)
    xs = xs_ref[...]
    g = jax.nn.sigmoid(
        jnp.dot(agg, wg[:HD, :], preferred_element_type=jnp.float32)
        + jnp.dot(xs, wg[HD:, :], preferred_element_type=jnp.float32) + bg[...])
    upd = agg + g * (s_ref[...] - agg)
    attn_out = jnp.dot(upd, wo[...], preferred_element_type=jnp.float32) + bo[...]
    out_ref[...] = x_ref[...] + _ln(attn_out, lpg[...], lpb[...])


def _post(aggu, xs, s, x, ffp, Wg, bg, Wo, bo, ln_post_g, ln_post_b):
    in_specs = [
        pl.BlockSpec((H // 2, NB, PW), lambda i: (0, i, 0)),
        _row_spec(),
        pl.BlockSpec((NB, HD), lambda i: (i, 0)),
        _row_spec(),
        _row_spec(),
        _full((HD + HID, HD)), _full((HD,)),
        _full((HD, HID)), _full((HID,)),
        _full((HID,)), _full((HID,)),
    ]
    out = pl.pallas_call(
        _post_body, grid=(GRID,), in_specs=in_specs,
        out_specs=_row_spec(),
        out_shape=jax.ShapeDtypeStruct((NP, HID), jnp.float32),
    )(aggu, xs, s, x, ffp, Wg, bg, Wo, bo, ln_post_g, ln_post_b)
    return out + ffp


def kernel(x, ln_src_g, ln_src_b, Wq, bq, Wk, Wv, bv, Ws, bs, Wg, bg, Wo, bo,
           ln_post_g, ln_post_b, ln_ffpre_g, ln_ffpre_b, W1, b1, W2, b2,
           ln_ffpost_g, ln_ffpost_b, edge_index):
    xp = jnp.pad(x, ((0, NP - N), (0, 0)))
    q, k, vt, s, xs, ffp = _pre(
        xp, ln_src_g, ln_src_b, Wq, bq, Wk, Wv, bv, Ws, bs,
        ln_ffpre_g, ln_ffpre_b, W1, b1, W2, b2, ln_ffpost_g, ln_ffpost_b)

    pad_idx = jnp.tile(jnp.arange(16, dtype=jnp.int32) + N, (EP - E) // 16)
    srcp = jnp.concatenate([edge_index[0], pad_idx])
    dstp = jnp.concatenate([edge_index[1], pad_idx])

    simflat, maxp = _sim_kernel(q, k, srcp, dstp)
    m = jnp.max(maxp, axis=(0, 2))
    mrep = jnp.broadcast_to(m[:, None], (H, 16)).reshape(H * 16)

    vt_flat = vt.reshape(H // 2 * NP, 2 * D)
    aggu = _agg_kernel(vt_flat, simflat, srcp, dstp, mrep)
    aggu = aggu.reshape(H // 2, NP, PW)

    out = _post(aggu, xs, s, xp, ffp, Wg, bg, Wo, bo, ln_post_g, ln_post_b)
    return out[:N]
